# baseline (speedup 1.0000x reference)
import functools

import jax
import jax.numpy as jnp
from jax import lax
from jax.experimental import pallas as pl
from jax.experimental.pallas import tpu as pltpu
from jax.experimental.pallas import tpu_sc as plsc

VOCAB = 1000000
EMBED_DIM = 64
BATCH = 4096
SEQ = 200

NC, NS, LANES = 2, 16, 16
NW = NC * NS
BPW = BATCH // NW
CH0, CH1 = 104, 96
NCH = BPW * 2
NRING = 4
GDEPTH = 2

_mesh = plsc.VectorSubcoreMesh(core_axis_name="c", subcore_axis_name="s")

_SPLAT_DNUMS = lax.GatherDimensionNumbers(
    offset_dims=(), collapsed_slice_dims=(0,), start_index_map=(0,))


def _splat(v, r):
    idx = jnp.full((16,), r, jnp.int32)
    return lax.gather(v, idx[:, None], _SPLAT_DNUMS, (1,),
                      mode=lax.GatherScatterMode.PROMISE_IN_BOUNDS)


@functools.partial(
    pl.kernel,
    mesh=_mesh,
    compiler_params=pltpu.CompilerParams(use_tc_tiling_on_sc=False),
    out_type=jax.ShapeDtypeStruct((BATCH, SEQ, EMBED_DIM), jnp.float32),
    scratch_types=[
        pltpu.VMEM((BPW, SEQ), jnp.int32),
        pltpu.VMEM((BPW, SEQ), jnp.int32),
        pltpu.VMEM((NRING, CH0, EMBED_DIM), jnp.float32),
        pltpu.SemaphoreType.DMA((NRING,)),
        pltpu.SemaphoreType.DMA((NRING,)),
    ],
)
def _emb_gather(x_hbm, m_hbm, table_hbm, out_hbm, idx_v, m_v, rows_v,
                gsem, wsem):
    wid = lax.axis_index("s") * NC + lax.axis_index("c")
    b0 = wid * BPW

    pltpu.sync_copy(x_hbm.at[pl.ds(b0, BPW), :], idx_v)
    pltpu.sync_copy(m_hbm.at[pl.ds(b0, BPW), :], m_v)

    def _geom(lc, h):
        bb = lc // 2
        start = 0 if h == 0 else CH0
        count = CH0 if h == 0 else CH1
        return bb, start, count

    def _gather(lc, h, s):
        bb, start, count = _geom(lc, h)
        return pltpu.make_async_copy(
            table_hbm.at[idx_v.at[bb, pl.ds(start, count)]],
            rows_v.at[s, pl.ds(0, count), :], gsem.at[s])

    def _write(lc, h, s):
        bb, start, count = _geom(lc, h)
        return pltpu.make_async_copy(
            rows_v.at[s, pl.ds(0, count), :],
            out_hbm.at[b0 + bb, pl.ds(start, count), :], wsem.at[s])

    for j in range(GDEPTH):
        _gather(j, j % 2, j % NRING).start()

    def _chunk(lc, h, s):
        bb, start, count = _geom(lc, h)
        _gather(lc, h, s).wait()

        def _mrows(goff, nrows):
            mvec = jnp.where(m_v[bb, pl.ds(start + goff, LANES)] > 0,
                             jnp.float32(1.0), jnp.float32(0.0))
            for r in range(nrows):
                sp = _splat(mvec, r)
                row = goff + r
                for kk in range(EMBED_DIM // LANES):
                    sl = pl.ds(kk * LANES, LANES)
                    rows_v[s, row, sl] = rows_v[s, row, sl] * sp

        def _mgroup(g, carry):
            _mrows(g * LANES, LANES)
            return carry

        lax.fori_loop(0, count // LANES, _mgroup, 0)
        if count % LANES:
            _mrows(count - count % LANES, count % LANES)

        _write(lc, h, s).start()

        @pl.when(lc + GDEPTH < NCH)
        def _prefetch():
            s2 = (lc + GDEPTH) % NRING

            @pl.when(lc >= NRING - GDEPTH)
            def _drain_prev_write():
                _write(lc - (NRING - GDEPTH), h, s2).wait()

            _gather(lc + GDEPTH, h, s2).start()

    def _outer(t, carry):
        for k in range(NRING):
            _chunk(t * NRING + k, k % 2, k)
        return carry

    lax.fori_loop(0, NCH // NRING, _outer, 0)

    for j in range(NCH - NRING, NCH):
        _write(j, j % 2, j % NRING).wait()


def kernel(x, mask, table):
    out = _emb_gather(x, mask, table)
    return out, mask


# speedup vs baseline: 1.0822x; 1.0822x over previous
import functools

import jax
import jax.numpy as jnp
from jax import lax
from jax.experimental import pallas as pl
from jax.experimental.pallas import tpu as pltpu
from jax.experimental.pallas import tpu_sc as plsc

VOCAB = 1000000
EMBED_DIM = 64
BATCH = 4096
SEQ = 200

NC, NS, LANES = 2, 16, 16
NW = NC * NS
CH = 128
ST = SEQ // 8
BT = BATCH // CH
NTRI = ST * BT * 8
NCH = NTRI // NW
NRING = 4
GDEPTH = 2

_mesh = plsc.VectorSubcoreMesh(core_axis_name="c", subcore_axis_name="s")

_SPLAT_DNUMS = lax.GatherDimensionNumbers(
    offset_dims=(), collapsed_slice_dims=(0,), start_index_map=(0,))


def _splat(v, r):
    idx = jnp.full((16,), r, jnp.int32)
    return lax.gather(v, idx[:, None], _SPLAT_DNUMS, (1,),
                      mode=lax.GatherScatterMode.PROMISE_IN_BOUNDS)


@functools.partial(
    pl.kernel,
    mesh=_mesh,
    compiler_params=pltpu.CompilerParams(use_tc_tiling_on_sc=False),
    out_type=jax.ShapeDtypeStruct((SEQ, BATCH, EMBED_DIM), jnp.float32),
    scratch_types=[
        pltpu.VMEM((NCH, CH), jnp.int32),
        pltpu.VMEM((NCH, CH), jnp.int32),
        pltpu.VMEM((NRING, CH, EMBED_DIM), jnp.float32),
        pltpu.SemaphoreType.DMA((NRING,)),
        pltpu.SemaphoreType.DMA((NRING,)),
    ],
)
def _emb_gather(x_hbm, m_hbm, table_hbm, out_hbm, idx_v, m_v, rows_v,
                gsem, wsem):
    wid = lax.axis_index("s") * NC + lax.axis_index("c")
    t0 = wid * NCH

    pltpu.sync_copy(x_hbm.at[pl.ds(t0, NCH), :], idx_v)
    pltpu.sync_copy(m_hbm.at[pl.ds(t0, NCH), :], m_v)

    def _geom(lc):
        g = t0 + lc
        ts = g // (BT * 8)
        rem = lax.rem(g, BT * 8)
        tb = rem // 8
        sr = lax.rem(rem, 8)
        return ts * 8 + sr, tb

    def _gather(lc, slot):
        return pltpu.make_async_copy(
            table_hbm.at[idx_v.at[lc, :]], rows_v.at[slot], gsem.at[slot])

    def _write(lc, slot):
        s, tb = _geom(lc)
        return pltpu.make_async_copy(
            rows_v.at[slot],
            out_hbm.at[s, pl.ds(tb * CH, CH), :], wsem.at[slot])

    for j in range(GDEPTH):
        _gather(j, j % NRING).start()

    def _chunk(lc, slot):
        _gather(lc, slot).wait()

        def _mgroup(g, carry):
            mvec = jnp.where(m_v[lc, pl.ds(g * LANES, LANES)] > 0,
                             jnp.float32(1.0), jnp.float32(0.0))
            for r in range(LANES):
                sp = _splat(mvec, r)
                row = g * LANES + r
                for kk in range(EMBED_DIM // LANES):
                    sl = pl.ds(kk * LANES, LANES)
                    rows_v[slot, row, sl] = rows_v[slot, row, sl] * sp
            return carry

        lax.fori_loop(0, CH // LANES, _mgroup, 0)

        _write(lc, slot).start()

        @pl.when(lc + GDEPTH < NCH)
        def _prefetch():
            s2 = (lc + GDEPTH) % NRING

            @pl.when(lc >= NRING - GDEPTH)
            def _drain_prev_write():
                _write(lc - (NRING - GDEPTH), s2).wait()

            _gather(lc + GDEPTH, s2).start()

    def _outer(t, carry):
        for k in range(NRING):
            _chunk(t * NRING + k, k)
        return carry

    lax.fori_loop(0, NCH // NRING, _outer, 0)

    for j in range(NCH - NRING, NCH):
        _write(j, j % NRING).wait()


def _tile_rows(a):
    return (a.T.reshape(ST, 8, BT, CH)
            .transpose(0, 2, 1, 3)
            .reshape(NTRI, CH))


def kernel(x, mask, table):
    out = _emb_gather(_tile_rows(x), _tile_rows(mask), table)
    return out.transpose(1, 0, 2), mask


# speedup vs baseline: 1.1242x; 1.0388x over previous
import functools

import jax
import jax.numpy as jnp
from jax import lax
from jax.experimental import pallas as pl
from jax.experimental.pallas import tpu as pltpu
from jax.experimental.pallas import tpu_sc as plsc

VOCAB = 1000000
EMBED_DIM = 64
BATCH = 4096
SEQ = 200

NC, NS, LANES = 2, 16, 16
NW = NC * NS
CH = 128
ST = SEQ // 8
BT = BATCH // CH
NTRI = ST * BT * 8
NCH = NTRI // NW
NRING = 5
GDEPTH = 3

_mesh = plsc.VectorSubcoreMesh(core_axis_name="c", subcore_axis_name="s")

_SPLAT_DNUMS = lax.GatherDimensionNumbers(
    offset_dims=(), collapsed_slice_dims=(0,), start_index_map=(0,))


def _splat(v, r):
    idx = jnp.full((16,), r, jnp.int32)
    return lax.gather(v, idx[:, None], _SPLAT_DNUMS, (1,),
                      mode=lax.GatherScatterMode.PROMISE_IN_BOUNDS)


@functools.partial(
    pl.kernel,
    mesh=_mesh,
    compiler_params=pltpu.CompilerParams(use_tc_tiling_on_sc=False),
    out_type=jax.ShapeDtypeStruct((SEQ, BATCH, EMBED_DIM), jnp.float32),
    scratch_types=[
        pltpu.VMEM((NCH, CH), jnp.int32),
        pltpu.VMEM((NCH, CH), jnp.int32),
        pltpu.VMEM((NRING, CH, EMBED_DIM), jnp.float32),
        pltpu.SemaphoreType.DMA((NRING,)),
        pltpu.SemaphoreType.DMA((NRING,)),
    ],
)
def _emb_gather(x_hbm, m_hbm, table_hbm, out_hbm, idx_v, m_v, rows_v,
                gsem, wsem):
    wid = lax.axis_index("s") * NC + lax.axis_index("c")
    t0 = wid * NCH

    pltpu.sync_copy(x_hbm.at[pl.ds(t0, NCH), :], idx_v)
    pltpu.sync_copy(m_hbm.at[pl.ds(t0, NCH), :], m_v)

    def _geom(lc):
        g = t0 + lc
        ts = g // (BT * 8)
        rem = lax.rem(g, BT * 8)
        tb = rem // 8
        sr = lax.rem(rem, 8)
        return ts * 8 + sr, tb

    def _gather(lc, slot):
        return pltpu.make_async_copy(
            table_hbm.at[idx_v.at[lc, :]], rows_v.at[slot], gsem.at[slot])

    def _write(lc, slot):
        s, tb = _geom(lc)
        return pltpu.make_async_copy(
            rows_v.at[slot],
            out_hbm.at[s, pl.ds(tb * CH, CH), :], wsem.at[slot])

    for j in range(GDEPTH):
        _gather(j, j % NRING).start()

    def _chunk(lc, slot):
        _gather(lc, slot).wait()

        def _mgroup(g, carry):
            mvec = jnp.where(m_v[lc, pl.ds(g * LANES, LANES)] > 0,
                             jnp.float32(1.0), jnp.float32(0.0))
            for r in range(LANES):
                sp = _splat(mvec, r)
                row = g * LANES + r
                for kk in range(EMBED_DIM // LANES):
                    sl = pl.ds(kk * LANES, LANES)
                    rows_v[slot, row, sl] = rows_v[slot, row, sl] * sp
            return carry

        lax.fori_loop(0, CH // LANES, _mgroup, 0)

        _write(lc, slot).start()

        @pl.when(lc + GDEPTH < NCH)
        def _prefetch():
            s2 = (lc + GDEPTH) % NRING

            @pl.when(lc >= NRING - GDEPTH)
            def _drain_prev_write():
                _write(lc - (NRING - GDEPTH), s2).wait()

            _gather(lc + GDEPTH, s2).start()

    def _outer(t, carry):
        for k in range(NRING):
            _chunk(t * NRING + k, k)
        return carry

    lax.fori_loop(0, NCH // NRING, _outer, 0)

    for j in range(NCH - NRING, NCH):
        _write(j, j % NRING).wait()


def _tile_rows(a):
    return (a.T.reshape(ST, 8, BT, CH)
            .transpose(0, 2, 1, 3)
            .reshape(NTRI, CH))


def kernel(x, mask, table):
    out = _emb_gather(_tile_rows(x), _tile_rows(mask), table)
    return out.transpose(1, 0, 2), mask


# speedup vs baseline: 1.1813x; 1.0508x over previous
import functools

import jax
import jax.numpy as jnp
from jax import lax
from jax.experimental import pallas as pl
from jax.experimental.pallas import tpu as pltpu
from jax.experimental.pallas import tpu_sc as plsc

VOCAB = 1000000
EMBED_DIM = 64
BATCH = 4096
SEQ = 200

NC, NS, LANES = 2, 16, 16
NW = NC * NS
CH = 128
ST = SEQ // 8
BT = BATCH // CH
NTRI = ST * BT * 8
NCH = NTRI // NW
NRING = 8
GDEPTH = 4

_mesh = plsc.VectorSubcoreMesh(core_axis_name="c", subcore_axis_name="s")

_SPLAT_DNUMS = lax.GatherDimensionNumbers(
    offset_dims=(), collapsed_slice_dims=(0,), start_index_map=(0,))


def _splat(v, r):
    idx = jnp.full((16,), r, jnp.int32)
    return lax.gather(v, idx[:, None], _SPLAT_DNUMS, (1,),
                      mode=lax.GatherScatterMode.PROMISE_IN_BOUNDS)


@functools.partial(
    pl.kernel,
    mesh=_mesh,
    compiler_params=pltpu.CompilerParams(use_tc_tiling_on_sc=False),
    out_type=jax.ShapeDtypeStruct((SEQ, BATCH, EMBED_DIM), jnp.float32),
    scratch_types=[
        pltpu.VMEM((NCH, CH), jnp.int32),
        pltpu.VMEM((NCH, CH), jnp.int32),
        pltpu.VMEM((NRING, CH, EMBED_DIM), jnp.float32),
        pltpu.SemaphoreType.DMA((NRING,)),
        pltpu.SemaphoreType.DMA((NRING,)),
    ],
)
def _emb_gather(x_hbm, m_hbm, table_hbm, out_hbm, idx_v, m_v, rows_v,
                gsem, wsem):
    wid = lax.axis_index("s") * NC + lax.axis_index("c")
    t0 = wid * NCH

    pltpu.sync_copy(x_hbm.at[pl.ds(t0, NCH), :], idx_v)
    pltpu.sync_copy(m_hbm.at[pl.ds(t0, NCH), :], m_v)

    def _geom(lc):
        g = t0 + lc
        ts = g // (BT * 8)
        rem = lax.rem(g, BT * 8)
        tb = rem // 8
        sr = lax.rem(rem, 8)
        return ts * 8 + sr, tb

    def _gather(lc, slot):
        return pltpu.make_async_copy(
            table_hbm.at[idx_v.at[lc, :]], rows_v.at[slot], gsem.at[slot])

    def _write(lc, slot):
        s, tb = _geom(lc)
        return pltpu.make_async_copy(
            rows_v.at[slot],
            out_hbm.at[s, pl.ds(tb * CH, CH), :], wsem.at[slot])

    for j in range(GDEPTH):
        _gather(j, j % NRING).start()

    def _chunk(lc, slot):
        _gather(lc, slot).wait()

        def _mgroup(g, carry):
            mvec = jnp.where(m_v[lc, pl.ds(g * LANES, LANES)] > 0,
                             jnp.float32(1.0), jnp.float32(0.0))
            for r in range(LANES):
                sp = _splat(mvec, r)
                row = g * LANES + r
                for kk in range(EMBED_DIM // LANES):
                    sl = pl.ds(kk * LANES, LANES)
                    rows_v[slot, row, sl] = rows_v[slot, row, sl] * sp
            return carry

        lax.fori_loop(0, CH // LANES, _mgroup, 0)

        _write(lc, slot).start()

        @pl.when(lc + GDEPTH < NCH)
        def _prefetch():
            s2 = (lc + GDEPTH) % NRING

            @pl.when(lc >= NRING - GDEPTH)
            def _drain_prev_write():
                _write(lc - (NRING - GDEPTH), s2).wait()

            _gather(lc + GDEPTH, s2).start()

    def _outer(t, carry):
        for k in range(NRING):
            _chunk(t * NRING + k, k)
        return carry

    lax.fori_loop(0, NCH // NRING, _outer, 0)

    for j in range(NCH - NRING, NCH):
        _write(j, j % NRING).wait()


def _tile_rows(a):
    return (a.T.reshape(ST, 8, BT, CH)
            .transpose(0, 2, 1, 3)
            .reshape(NTRI, CH))


def kernel(x, mask, table):
    out = _emb_gather(_tile_rows(x), _tile_rows(mask), table)
    return out.transpose(1, 0, 2), mask


# speedup vs baseline: 1.1898x; 1.0072x over previous
import functools

import jax
import jax.numpy as jnp
from jax import lax
from jax.experimental import pallas as pl
from jax.experimental.pallas import tpu as pltpu
from jax.experimental.pallas import tpu_sc as plsc

VOCAB = 1000000
EMBED_DIM = 64
BATCH = 4096
SEQ = 200

NC, NS, LANES = 2, 16, 16
NW = NC * NS
CH = 128
ST = SEQ // 8
BT = BATCH // CH
NTRI = ST * BT * 8
NCH = NTRI // NW
NRING = 8
GDEPTH = 5

_mesh = plsc.VectorSubcoreMesh(core_axis_name="c", subcore_axis_name="s")

_SPLAT_DNUMS = lax.GatherDimensionNumbers(
    offset_dims=(), collapsed_slice_dims=(0,), start_index_map=(0,))


def _splat(v, r):
    idx = jnp.full((16,), r, jnp.int32)
    return lax.gather(v, idx[:, None], _SPLAT_DNUMS, (1,),
                      mode=lax.GatherScatterMode.PROMISE_IN_BOUNDS)


@functools.partial(
    pl.kernel,
    mesh=_mesh,
    compiler_params=pltpu.CompilerParams(use_tc_tiling_on_sc=False),
    out_type=jax.ShapeDtypeStruct((SEQ, BATCH, EMBED_DIM), jnp.float32),
    scratch_types=[
        pltpu.VMEM((NCH, CH), jnp.int32),
        pltpu.VMEM((NCH, CH), jnp.int32),
        pltpu.VMEM((NRING, CH, EMBED_DIM), jnp.float32),
        pltpu.SemaphoreType.DMA((NRING,)),
        pltpu.SemaphoreType.DMA((NRING,)),
    ],
)
def _emb_gather(x_hbm, m_hbm, table_hbm, out_hbm, idx_v, m_v, rows_v,
                gsem, wsem):
    wid = lax.axis_index("s") * NC + lax.axis_index("c")
    t0 = wid * NCH

    pltpu.sync_copy(x_hbm.at[pl.ds(t0, NCH), :], idx_v)
    pltpu.sync_copy(m_hbm.at[pl.ds(t0, NCH), :], m_v)

    def _geom(lc):
        g = t0 + lc
        ts = g // (BT * 8)
        rem = lax.rem(g, BT * 8)
        tb = rem // 8
        sr = lax.rem(rem, 8)
        return ts * 8 + sr, tb

    def _gather(lc, slot):
        return pltpu.make_async_copy(
            table_hbm.at[idx_v.at[lc, :]], rows_v.at[slot], gsem.at[slot])

    def _write(lc, slot):
        s, tb = _geom(lc)
        return pltpu.make_async_copy(
            rows_v.at[slot],
            out_hbm.at[s, pl.ds(tb * CH, CH), :], wsem.at[slot])

    for j in range(GDEPTH):
        _gather(j, j % NRING).start()

    def _chunk(lc, slot):
        _gather(lc, slot).wait()

        def _mgroup(g, carry):
            mvec = jnp.where(m_v[lc, pl.ds(g * LANES, LANES)] > 0,
                             jnp.float32(1.0), jnp.float32(0.0))
            for r in range(LANES):
                sp = _splat(mvec, r)
                row = g * LANES + r
                for kk in range(EMBED_DIM // LANES):
                    sl = pl.ds(kk * LANES, LANES)
                    rows_v[slot, row, sl] = rows_v[slot, row, sl] * sp
            return carry

        lax.fori_loop(0, CH // LANES, _mgroup, 0)

        _write(lc, slot).start()

        @pl.when(lc + GDEPTH < NCH)
        def _prefetch():
            s2 = (lc + GDEPTH) % NRING

            @pl.when(lc >= NRING - GDEPTH)
            def _drain_prev_write():
                _write(lc - (NRING - GDEPTH), s2).wait()

            _gather(lc + GDEPTH, s2).start()

    def _outer(t, carry):
        for k in range(NRING):
            _chunk(t * NRING + k, k)
        return carry

    lax.fori_loop(0, NCH // NRING, _outer, 0)

    for j in range(NCH - NRING, NCH):
        _write(j, j % NRING).wait()


def _tile_rows(a):
    return (a.T.reshape(ST, 8, BT, CH)
            .transpose(0, 2, 1, 3)
            .reshape(NTRI, CH))


def kernel(x, mask, table):
    out = _emb_gather(_tile_rows(x), _tile_rows(mask), table)
    return out.transpose(1, 0, 2), mask
